# baseline (speedup 1.0000x reference)
import functools

import jax
import jax.numpy as jnp
from jax import lax
from jax.experimental import pallas as pl
from jax.experimental.pallas import tpu as pltpu
from jax.experimental.pallas import tpu_sc as plsc

_N = 100000
_D = 768
_Q = 1024
_K = 32
_C = 1024
_NN = (_N + _C - 1) // _C
_NEG = float("-inf")
_IMAX = 2**31 - 1


def _topk_body(q_ref, x_ref, ov_ref, oi_ref, s_ref, rv_ref, ri_ref, qs_ref):
    n = pl.program_id(0)

    @pl.when(n == 0)
    def _init():
        rv_ref[...] = jnp.full((_Q, _K), _NEG, jnp.float32)
        ri_ref[...] = jnp.zeros((_Q, _K), jnp.int32)
        q = q_ref[...]
        qs_ref[...] = jnp.sum(q * q, axis=1, keepdims=True)

    x = x_ref[...]
    xsq = jnp.sum(x * x, axis=1)
    m = lax.dot_general(q_ref[...], x, (((1,), (1,)), ((), ())),
                        preferred_element_type=jnp.float32)
    d2 = (qs_ref[...] + xsq[None, :]) - 2.0 * m
    col = lax.broadcasted_iota(jnp.int32, (_Q, _C), 1) + n * _C
    s = jnp.where(col < _N, -d2, _NEG)
    s_ref[...] = s

    t = jnp.min(rv_ref[...], axis=1, keepdims=True)
    c = jnp.sum((s > t).astype(jnp.int32), axis=1, keepdims=True)
    n_it = jnp.max(jnp.minimum(c, _K))

    lane = lax.broadcasted_iota(jnp.int32, (_Q, _K), 1)

    def insert(v, a):
        rv = rv_ref[...]
        ri = ri_ref[...]
        pos = jnp.sum((rv >= v).astype(jnp.int32), axis=1, keepdims=True)
        rvs = jnp.concatenate([rv[:, :1], rv[:, :_K - 1]], axis=1)
        ris = jnp.concatenate([ri[:, :1], ri[:, :_K - 1]], axis=1)
        rv_ref[...] = jnp.where(lane < pos, rv,
                                jnp.where(lane == pos, v, rvs))
        ri_ref[...] = jnp.where(lane < pos, ri,
                                jnp.where(lane == pos, a, ris))

    def body(i, carry):
        sc = s_ref[...]
        v1 = jnp.max(sc, axis=1, keepdims=True)
        a1 = jnp.min(jnp.where(sc == v1, col, _IMAX), axis=1, keepdims=True)
        sc = jnp.where(col == a1, _NEG, sc)
        v2 = jnp.max(sc, axis=1, keepdims=True)
        a2 = jnp.min(jnp.where(sc == v2, col, _IMAX), axis=1, keepdims=True)
        s_ref[...] = jnp.where(col == a2, _NEG, sc)
        insert(v1, a1)
        insert(v2, a2)
        return carry

    lax.fori_loop(0, (n_it + 1) // 2, body, 0)

    @pl.when(n == _NN - 1)
    def _out():
        ov_ref[...] = rv_ref[...]
        oi_ref[...] = ri_ref[...]


def _topk(queries, x_train):
    return pl.pallas_call(
        _topk_body,
        grid=(_NN,),
        in_specs=[
            pl.BlockSpec((_Q, _D), lambda n: (0, 0)),
            pl.BlockSpec((_C, _D), lambda n: (n, 0)),
        ],
        out_specs=[
            pl.BlockSpec((_Q, _K), lambda n: (0, 0)),
            pl.BlockSpec((_Q, _K), lambda n: (0, 0)),
        ],
        out_shape=[
            jax.ShapeDtypeStruct((_Q, _K), jnp.float32),
            jax.ShapeDtypeStruct((_Q, _K), jnp.int32),
        ],
        scratch_shapes=[
            pltpu.VMEM((_Q, _C), jnp.float32),
            pltpu.VMEM((_Q, _K), jnp.float32),
            pltpu.VMEM((_Q, _K), jnp.int32),
            pltpu.VMEM((_Q, 1), jnp.float32),
        ],
        compiler_params=pltpu.CompilerParams(
            dimension_semantics=("arbitrary",),
            vmem_limit_bytes=100 * 1024 * 1024),
    )(queries, x_train)


def _sc_gather(tables, idx_flat):
    b = idx_flat.shape[0]
    per = b // 16
    mesh = plsc.VectorSubcoreMesh(core_axis_name="c", subcore_axis_name="s")

    @functools.partial(
        pl.kernel, mesh=mesh,
        out_type=jax.ShapeDtypeStruct((2, b), jnp.int32),
        scratch_types=[
            pltpu.VMEM((_N,), jnp.int32),
            pltpu.VMEM((per,), jnp.int32),
            pltpu.VMEM((per,), jnp.int32),
        ],
        compiler_params=pltpu.CompilerParams(needs_layout_passes=False),
    )
    def k(tab_hbm, i_hbm, o_hbm, tab_v, idx_v, out_v):
        c = lax.axis_index("c")
        s = lax.axis_index("s")
        base = s * per
        pltpu.sync_copy(i_hbm.at[pl.ds(base, per)], idx_v)
        pltpu.sync_copy(tab_hbm.at[c], tab_v)

        def body(j, carry):
            iv = idx_v[pl.ds(j * 16, 16)]
            out_v[pl.ds(j * 16, 16)] = plsc.load_gather(tab_v, [iv])
            return carry

        lax.fori_loop(0, per // 16, body, 0)
        pltpu.sync_copy(out_v, o_hbm.at[c, pl.ds(base, per)])

    return k(tables, idx_flat)


_QB = 128


def _fin_body(sv_ref, kv_ref, rs_ref, qs_ref, p_ref, od_ref, os_ref):
    dist = -sv_ref[...]
    w0 = p_ref[0, 0]
    w1 = p_ref[0, 1]
    b0 = p_ref[0, 2]
    b1 = p_ref[0, 3]
    local = rs_ref[...] == qs_ref[...]
    nd = jnp.where(local, dist * w1 + b1, dist * w0 + b0)
    od_ref[...] = nd
    z = -nd
    v = kv_ref[...]
    lane = lax.broadcasted_iota(jnp.int32, (_QB, _K), 1)
    cols = []
    for j in range(_K):
        msk = lane <= j
        zj = jnp.where(msk, z, _NEG)
        mj = jnp.max(zj, axis=1, keepdims=True)
        ej = jnp.where(msk, jnp.exp(z - mj), 0.0)
        cols.append(jnp.sum(ej * v, axis=1, keepdims=True) /
                    jnp.sum(ej, axis=1, keepdims=True))
    os_ref[...] = jnp.concatenate(cols, axis=1)


def _finalize(svals, knn_vals, res_sys, qsys, params):
    return pl.pallas_call(
        _fin_body,
        grid=(_Q // _QB,),
        in_specs=[
            pl.BlockSpec((_QB, _K), lambda i: (i, 0)),
            pl.BlockSpec((_QB, _K), lambda i: (i, 0)),
            pl.BlockSpec((_QB, _K), lambda i: (i, 0)),
            pl.BlockSpec((_QB, 1), lambda i: (i, 0)),
            pl.BlockSpec(memory_space=pltpu.SMEM),
        ],
        out_specs=[
            pl.BlockSpec((_QB, _K), lambda i: (i, 0)),
            pl.BlockSpec((_QB, _K), lambda i: (i, 0)),
        ],
        out_shape=[
            jax.ShapeDtypeStruct((_Q, _K), jnp.float32),
            jax.ShapeDtypeStruct((_Q, _K), jnp.float32),
        ],
    )(svals, knn_vals, res_sys, qsys, params)


def kernel(queries, query_sys, X_train, Y_train, sys_train, W_local, b_local):
    svals, sidx = _topk(queries, X_train)
    tables = jnp.stack([lax.bitcast_convert_type(Y_train, jnp.int32),
                        sys_train.astype(jnp.int32)])
    gathered = _sc_gather(tables, sidx.reshape(-1))
    knn_vals = lax.bitcast_convert_type(gathered[0], jnp.float32).reshape(
        _Q, _K)
    res_sys = gathered[1].reshape(_Q, _K)
    qsys = query_sys.astype(jnp.int32).reshape(_Q, 1)
    params = jnp.concatenate(
        [W_local.reshape(2), b_local.reshape(2)]).reshape(1, 4)
    return _finalize(svals, knn_vals, res_sys, qsys, params)


# speedup vs baseline: 1.0562x; 1.0562x over previous
import functools

import jax
import jax.numpy as jnp
from jax import lax
from jax.experimental import pallas as pl
from jax.experimental.pallas import tpu as pltpu
from jax.experimental.pallas import tpu_sc as plsc

_N = 100000
_D = 768
_Q = 1024
_K = 32
_C = 1024
_NN = (_N + _C - 1) // _C
_NEG = float("-inf")
_IMAX = 2**31 - 1


def _topk_body(q_ref, x_ref, ov_ref, oi_ref, s_ref, rv_ref, ri_ref, qs_ref):
    n = pl.program_id(0)

    @pl.when(n == 0)
    def _init():
        rv_ref[...] = jnp.full((_Q, _K), _NEG, jnp.float32)
        ri_ref[...] = jnp.zeros((_Q, _K), jnp.int32)
        q = q_ref[...]
        qs_ref[...] = jnp.sum(q * q, axis=1, keepdims=True)

    x = x_ref[...]
    xsq = jnp.sum(x * x, axis=1)
    m = lax.dot_general(q_ref[...], x, (((1,), (1,)), ((), ())),
                        preferred_element_type=jnp.float32)
    d2 = (qs_ref[...] + xsq[None, :]) - 2.0 * m
    col = lax.broadcasted_iota(jnp.int32, (_Q, _C), 1) + n * _C
    s = jnp.where(col < _N, -d2, _NEG)
    s_ref[...] = s

    t = jnp.min(rv_ref[...], axis=1, keepdims=True)
    c = jnp.sum((s > t).astype(jnp.int32), axis=1, keepdims=True)
    n_it = jnp.max(jnp.minimum(c, _K))

    lane = lax.broadcasted_iota(jnp.int32, (_Q, _K), 1)

    def insert(v, a):
        rv = rv_ref[...]
        ri = ri_ref[...]
        pos = jnp.sum((rv >= v).astype(jnp.int32), axis=1, keepdims=True)
        rvs = jnp.concatenate([rv[:, :1], rv[:, :_K - 1]], axis=1)
        ris = jnp.concatenate([ri[:, :1], ri[:, :_K - 1]], axis=1)
        rv_ref[...] = jnp.where(lane < pos, rv,
                                jnp.where(lane == pos, v, rvs))
        ri_ref[...] = jnp.where(lane < pos, ri,
                                jnp.where(lane == pos, a, ris))

    def body(i, carry):
        sc = s_ref[...]
        v1 = jnp.max(sc, axis=1, keepdims=True)
        a1 = jnp.min(jnp.where(sc == v1, col, _IMAX), axis=1, keepdims=True)
        sc = jnp.where(col == a1, _NEG, sc)
        v2 = jnp.max(sc, axis=1, keepdims=True)
        a2 = jnp.min(jnp.where(sc == v2, col, _IMAX), axis=1, keepdims=True)
        s_ref[...] = jnp.where(col == a2, _NEG, sc)
        insert(v1, a1)
        insert(v2, a2)
        return carry

    lax.fori_loop(0, n_it // 2, body, 0)

    @pl.when(n_it % 2 == 1)
    def _odd():
        sc = s_ref[...]
        v = jnp.max(sc, axis=1, keepdims=True)
        a = jnp.min(jnp.where(sc == v, col, _IMAX), axis=1, keepdims=True)
        s_ref[...] = jnp.where(col == a, _NEG, sc)
        insert(v, a)

    @pl.when(n == _NN - 1)
    def _out():
        ov_ref[...] = rv_ref[...]
        oi_ref[...] = ri_ref[...]


def _topk(queries, x_train):
    return pl.pallas_call(
        _topk_body,
        grid=(_NN,),
        in_specs=[
            pl.BlockSpec((_Q, _D), lambda n: (0, 0)),
            pl.BlockSpec((_C, _D), lambda n: (n, 0)),
        ],
        out_specs=[
            pl.BlockSpec((_Q, _K), lambda n: (0, 0)),
            pl.BlockSpec((_Q, _K), lambda n: (0, 0)),
        ],
        out_shape=[
            jax.ShapeDtypeStruct((_Q, _K), jnp.float32),
            jax.ShapeDtypeStruct((_Q, _K), jnp.int32),
        ],
        scratch_shapes=[
            pltpu.VMEM((_Q, _C), jnp.float32),
            pltpu.VMEM((_Q, _K), jnp.float32),
            pltpu.VMEM((_Q, _K), jnp.int32),
            pltpu.VMEM((_Q, 1), jnp.float32),
        ],
        compiler_params=pltpu.CompilerParams(
            dimension_semantics=("arbitrary",),
            vmem_limit_bytes=100 * 1024 * 1024),
    )(queries, x_train)


def _sc_gather(tables, idx_flat):
    b = idx_flat.shape[0]
    per = b // 16
    mesh = plsc.VectorSubcoreMesh(core_axis_name="c", subcore_axis_name="s")

    @functools.partial(
        pl.kernel, mesh=mesh,
        out_type=jax.ShapeDtypeStruct((2, b), jnp.int32),
        scratch_types=[
            pltpu.VMEM((_N,), jnp.int32),
            pltpu.VMEM((per,), jnp.int32),
            pltpu.VMEM((per,), jnp.int32),
        ],
        compiler_params=pltpu.CompilerParams(needs_layout_passes=False),
    )
    def k(tab_hbm, i_hbm, o_hbm, tab_v, idx_v, out_v):
        c = lax.axis_index("c")
        s = lax.axis_index("s")
        base = s * per
        pltpu.sync_copy(i_hbm.at[pl.ds(base, per)], idx_v)
        pltpu.sync_copy(tab_hbm.at[c], tab_v)

        def body(j, carry):
            iv = idx_v[pl.ds(j * 16, 16)]
            out_v[pl.ds(j * 16, 16)] = plsc.load_gather(tab_v, [iv])
            return carry

        lax.fori_loop(0, per // 16, body, 0)
        pltpu.sync_copy(out_v, o_hbm.at[c, pl.ds(base, per)])

    return k(tables, idx_flat)


_QB = 128


def _fin_body(sv_ref, kv_ref, rs_ref, qs_ref, p_ref, od_ref, os_ref):
    dist = -sv_ref[...]
    w0 = p_ref[0, 0]
    w1 = p_ref[0, 1]
    b0 = p_ref[0, 2]
    b1 = p_ref[0, 3]
    local = rs_ref[...] == qs_ref[...]
    nd = jnp.where(local, dist * w1 + b1, dist * w0 + b0)
    od_ref[...] = nd
    z = -nd
    v = kv_ref[...]
    lane = lax.broadcasted_iota(jnp.int32, (_QB, _K), 1)
    cols = []
    for j in range(_K):
        msk = lane <= j
        zj = jnp.where(msk, z, _NEG)
        mj = jnp.max(zj, axis=1, keepdims=True)
        ej = jnp.where(msk, jnp.exp(z - mj), 0.0)
        cols.append(jnp.sum(ej * v, axis=1, keepdims=True) /
                    jnp.sum(ej, axis=1, keepdims=True))
    os_ref[...] = jnp.concatenate(cols, axis=1)


def _finalize(svals, knn_vals, res_sys, qsys, params):
    return pl.pallas_call(
        _fin_body,
        grid=(_Q // _QB,),
        in_specs=[
            pl.BlockSpec((_QB, _K), lambda i: (i, 0)),
            pl.BlockSpec((_QB, _K), lambda i: (i, 0)),
            pl.BlockSpec((_QB, _K), lambda i: (i, 0)),
            pl.BlockSpec((_QB, 1), lambda i: (i, 0)),
            pl.BlockSpec(memory_space=pltpu.SMEM),
        ],
        out_specs=[
            pl.BlockSpec((_QB, _K), lambda i: (i, 0)),
            pl.BlockSpec((_QB, _K), lambda i: (i, 0)),
        ],
        out_shape=[
            jax.ShapeDtypeStruct((_Q, _K), jnp.float32),
            jax.ShapeDtypeStruct((_Q, _K), jnp.float32),
        ],
    )(svals, knn_vals, res_sys, qsys, params)


def kernel(queries, query_sys, X_train, Y_train, sys_train, W_local, b_local):
    svals, sidx = _topk(queries, X_train)
    tables = jnp.stack([lax.bitcast_convert_type(Y_train, jnp.int32),
                        sys_train.astype(jnp.int32)])
    gathered = _sc_gather(tables, sidx.reshape(-1))
    knn_vals = lax.bitcast_convert_type(gathered[0], jnp.float32).reshape(
        _Q, _K)
    res_sys = gathered[1].reshape(_Q, _K)
    qsys = query_sys.astype(jnp.int32).reshape(_Q, 1)
    params = jnp.concatenate(
        [W_local.reshape(2), b_local.reshape(2)]).reshape(1, 4)
    return _finalize(svals, knn_vals, res_sys, qsys, params)
